# Initial kernel scaffold; baseline (speedup 1.0000x reference)
import jax
import jax.numpy as jnp
from jax.experimental import pallas as pl


def kernel(tensor, edge_index, W1, b1, W2, b2):
    raise NotImplementedError("write your pallas kernel here")


# speedup vs baseline: 17.0541x; 17.0541x over previous
import functools

import jax
import jax.numpy as jnp
from jax import lax
from jax.experimental import pallas as pl
from jax.experimental.pallas import tpu as pltpu
from jax.experimental.pallas import tpu_sc as plsc

N = 10000
NP = 10240
IN_CH = 128
HID = 256
OUT = 128
NC = 2
NS = 16
LANES = 16

_MESH = dict(core_axis_name="c", subcore_axis_name="s")


def _deg_call(dst32):
    epw = dst32.shape[1]

    @functools.partial(
        pl.kernel,
        out_type=jax.ShapeDtypeStruct((NC * NS, NP), jnp.float32),
        mesh=plsc.VectorSubcoreMesh(**_MESH),
        compiler_params=pltpu.CompilerParams(needs_layout_passes=False),
        scratch_types=[
            pltpu.VMEM((epw,), jnp.int32),
            pltpu.VMEM((NP,), jnp.float32),
        ],
    )
    def k(dst_hbm, out_hbm, dbuf, cnt):
        cid = lax.axis_index("c")
        sid = lax.axis_index("s")
        wid = sid * NC + cid
        pltpu.sync_copy(dst_hbm.at[wid], dbuf)
        zeros = jnp.zeros((LANES,), jnp.float32)

        def zbody(i, c):
            cnt[pl.ds(pl.multiple_of(i * LANES, LANES), LANES)] = zeros
            return c

        lax.fori_loop(0, NP // LANES, zbody, 0)
        ones = jnp.full((LANES,), 1.0, jnp.float32)

        def body(i, c):
            idx = dbuf[pl.ds(pl.multiple_of(i * LANES, LANES), LANES)]
            plsc.addupdate_scatter(cnt, [idx], ones)
            return c

        lax.fori_loop(0, epw // LANES, body, 0)
        pltpu.sync_copy(cnt, out_hbm.at[wid])

    return k(dst32)


_KB = 8


def _agg_call(h0, h1, src4, dst4, zin):
    D = h0.shape[1]
    nblk = src4.shape[1]
    rps = NP // NS

    @functools.partial(
        pl.kernel,
        out_type=[jax.ShapeDtypeStruct((NP, D), jnp.float32)] * 2,
        mesh=plsc.VectorSubcoreMesh(**_MESH),
        compiler_params=pltpu.CompilerParams(needs_layout_passes=False),
        scratch_types=[
            pltpu.VMEM((_KB, 128), jnp.int32),
            pltpu.VMEM((_KB, 128), jnp.int32),
            pltpu.VMEM((128, D), jnp.float32),
            pltpu.VMEM_SHARED((NP, D), jnp.float32),
            pltpu.SemaphoreType.DMA,
        ],
    )
    def k(h0_hbm, h1_hbm, src_hbm, dst_hbm, z_hbm, a0_hbm, a1_hbm,
          sbuf, dbuf, rows, acc, gsem):
        cid = lax.axis_index("c")
        sid = lax.axis_index("s")

        for t in range(rps // 128):
            pltpu.sync_copy(z_hbm, acc.at[pl.ds(sid * rps + t * 128, 128)])
        plsc.subcore_barrier()

        def body(j, c):
            pltpu.sync_copy(src_hbm.at[sid, j], sbuf)
            pltpu.sync_copy(dst_hbm.at[sid, j], dbuf)
            for t in range(_KB):
                @pl.when(cid == 0)
                def _():
                    pltpu.async_copy(
                        h0_hbm.at[sbuf.at[t]], rows, gsem).wait()

                @pl.when(cid == 1)
                def _():
                    pltpu.async_copy(
                        h1_hbm.at[sbuf.at[t]], rows, gsem).wait()

                pltpu.sync_copy(rows, acc.at[dbuf.at[t]], add=True)
            return c

        lax.fori_loop(0, nblk, body, 0)
        plsc.subcore_barrier()

        sl = pl.ds(sid * rps, rps)

        @pl.when(cid == 0)
        def _():
            pltpu.sync_copy(acc.at[sl], a0_hbm.at[sl])

        @pl.when(cid == 1)
        def _():
            pltpu.sync_copy(acc.at[sl], a1_hbm.at[sl])

    return k(h0, h1, src4, dst4, zin)


def _agg2_call(h, src4, dst4, zin):
    D = h.shape[1]
    nblk = src4.shape[1]
    nhalf = nblk // 2
    rps = NP // NS

    @functools.partial(
        pl.kernel,
        out_type=[jax.ShapeDtypeStruct((NP, D), jnp.float32)] * 2,
        mesh=plsc.VectorSubcoreMesh(**_MESH),
        compiler_params=pltpu.CompilerParams(needs_layout_passes=False),
        scratch_types=[
            pltpu.VMEM((_KB, 128), jnp.int32),
            pltpu.VMEM((_KB, 128), jnp.int32),
            pltpu.VMEM((128, D), jnp.float32),
            pltpu.VMEM_SHARED((NP, D), jnp.float32),
            pltpu.SemaphoreType.DMA,
        ],
    )
    def k(h_hbm, src_hbm, dst_hbm, z_hbm, a0_hbm, a1_hbm,
          sbuf, dbuf, rows, acc, gsem):
        cid = lax.axis_index("c")
        sid = lax.axis_index("s")

        for t in range(rps // 128):
            pltpu.sync_copy(z_hbm, acc.at[pl.ds(sid * rps + t * 128, 128)])
        plsc.subcore_barrier()

        def body(j, c):
            blk = cid * nhalf + j
            pltpu.sync_copy(src_hbm.at[sid, blk], sbuf)
            pltpu.sync_copy(dst_hbm.at[sid, blk], dbuf)
            for t in range(_KB):
                pltpu.async_copy(h_hbm.at[sbuf.at[t]], rows, gsem).wait()
                pltpu.sync_copy(rows, acc.at[dbuf.at[t]], add=True)
            return c

        lax.fori_loop(0, nhalf, body, 0)
        plsc.subcore_barrier()

        sl = pl.ds(sid * rps, rps)

        @pl.when(cid == 0)
        def _():
            pltpu.sync_copy(acc.at[sl], a0_hbm.at[sl])

        @pl.when(cid == 1)
        def _():
            pltpu.sync_copy(acc.at[sl], a1_hbm.at[sl])

    return k(h, src4, dst4, zin)


_NB = 8
_BR = NP // _NB


def _dinv_of(cnt_blk):
    deg = jnp.sum(cnt_blk, axis=0) + 1.0
    return lax.rsqrt(deg)


def _mm1_call(x_pad, cnt, W1):
    def body(x_ref, cnt_ref, w_ref, h0_ref, h1_ref):
        dinv = _dinv_of(cnt_ref[...])
        h = jnp.dot(x_ref[...], w_ref[...], preferred_element_type=jnp.float32)
        h = h * dinv[:, None]
        h0_ref[...] = h[:, : HID // 2]
        h1_ref[...] = h[:, HID // 2:]

    return pl.pallas_call(
        body,
        grid=(_NB,),
        in_specs=[
            pl.BlockSpec((_BR, IN_CH), lambda i: (i, 0)),
            pl.BlockSpec((NC * NS, _BR), lambda i: (0, i)),
            pl.BlockSpec((IN_CH, HID), lambda i: (0, 0)),
        ],
        out_specs=[
            pl.BlockSpec((_BR, HID // 2), lambda i: (i, 0)),
            pl.BlockSpec((_BR, HID // 2), lambda i: (i, 0)),
        ],
        out_shape=[jax.ShapeDtypeStruct((NP, HID // 2), jnp.float32)] * 2,
    )(x_pad, cnt, W1)


def _mid_call(a0, a1, h0, h1, cnt, W2, b1):
    def body(a0_ref, a1_ref, h0_ref, h1_ref, cnt_ref, w_ref, b_ref, o_ref):
        dinv = _dinv_of(cnt_ref[...])
        x2 = jnp.concatenate(
            [a0_ref[...] + h0_ref[...], a1_ref[...] + h1_ref[...]], axis=1)
        x2 = jnp.maximum(x2 * dinv[:, None] + b_ref[...], 0.0)
        h2 = jnp.dot(x2, w_ref[...], preferred_element_type=jnp.float32)
        o_ref[...] = h2 * dinv[:, None]

    return pl.pallas_call(
        body,
        grid=(_NB,),
        in_specs=[
            pl.BlockSpec((_BR, HID // 2), lambda i: (i, 0)),
            pl.BlockSpec((_BR, HID // 2), lambda i: (i, 0)),
            pl.BlockSpec((_BR, HID // 2), lambda i: (i, 0)),
            pl.BlockSpec((_BR, HID // 2), lambda i: (i, 0)),
            pl.BlockSpec((NC * NS, _BR), lambda i: (0, i)),
            pl.BlockSpec((HID, OUT), lambda i: (0, 0)),
            pl.BlockSpec((1, HID), lambda i: (0, 0)),
        ],
        out_specs=pl.BlockSpec((_BR, OUT), lambda i: (i, 0)),
        out_shape=jax.ShapeDtypeStruct((NP, OUT), jnp.float32),
    )(a0, a1, h0, h1, cnt, W2, b1)


def _final_call(p0, p1, h2, cnt, b2):
    def body(p0_ref, p1_ref, h2_ref, cnt_ref, b_ref, o_ref):
        dinv = _dinv_of(cnt_ref[...])
        t = p0_ref[...] + p1_ref[...] + h2_ref[...]
        o_ref[...] = jnp.maximum(t * dinv[:, None] + b_ref[...], 0.0)

    return pl.pallas_call(
        body,
        grid=(_NB,),
        in_specs=[
            pl.BlockSpec((_BR, OUT), lambda i: (i, 0)),
            pl.BlockSpec((_BR, OUT), lambda i: (i, 0)),
            pl.BlockSpec((_BR, OUT), lambda i: (i, 0)),
            pl.BlockSpec((NC * NS, _BR), lambda i: (0, i)),
            pl.BlockSpec((1, OUT), lambda i: (0, 0)),
        ],
        out_specs=pl.BlockSpec((_BR, OUT), lambda i: (i, 0)),
        out_shape=jax.ShapeDtypeStruct((NP, OUT), jnp.float32),
    )(p0, p1, h2, cnt, b2)


def kernel(tensor, edge_index, W1, b1, W2, b2):
    e = edge_index.shape[1]
    gran = NS * _KB * 128
    ep = -(-e // gran) * gran
    npad = ep - e
    ei = edge_index.astype(jnp.int32)
    ar = jnp.arange(npad, dtype=jnp.int32)
    src_p = jnp.concatenate([ei[0], ar % N])
    dst_p = jnp.concatenate([ei[1], N + ar % (NP - N)])
    nblk = ep // NS // (_KB * 128)
    src4 = src_p.reshape(NS, nblk, _KB, 128)
    dst4 = dst_p.reshape(NS, nblk, _KB, 128)
    dst32 = dst_p.reshape(NC * NS, ep // (NC * NS))
    x_pad = jnp.pad(tensor, ((0, NP - N), (0, 0)))
    z128 = jnp.zeros((128, 128), jnp.float32)

    cnt = _deg_call(dst32)
    h1a, h1b = _mm1_call(x_pad, cnt, W1)
    a1a, a1b = _agg_call(h1a, h1b, src4, dst4, z128)
    h2 = _mid_call(a1a, a1b, h1a, h1b, cnt, W2, b1.reshape(1, -1))
    p0, p1 = _agg2_call(h2, src4, dst4, z128)
    out = _final_call(p0, p1, h2, cnt, b2.reshape(1, -1))
    return out[:N]


# speedup vs baseline: 27.4885x; 1.6118x over previous
import functools

import jax
import jax.numpy as jnp
from jax import lax
from jax.experimental import pallas as pl
from jax.experimental.pallas import tpu as pltpu
from jax.experimental.pallas import tpu_sc as plsc

N = 10000
NP = 10240
IN_CH = 128
HID = 256
OUT = 128
NC = 2
NS = 16
LANES = 16

_MESH = dict(core_axis_name="c", subcore_axis_name="s")


def _deg_call(dst32):
    epw = dst32.shape[1]

    @functools.partial(
        pl.kernel,
        out_type=jax.ShapeDtypeStruct((NC * NS, NP), jnp.float32),
        mesh=plsc.VectorSubcoreMesh(**_MESH),
        compiler_params=pltpu.CompilerParams(needs_layout_passes=False),
        scratch_types=[
            pltpu.VMEM((epw,), jnp.int32),
            pltpu.VMEM((NP,), jnp.float32),
        ],
    )
    def k(dst_hbm, out_hbm, dbuf, cnt):
        cid = lax.axis_index("c")
        sid = lax.axis_index("s")
        wid = sid * NC + cid
        pltpu.sync_copy(dst_hbm.at[wid], dbuf)
        zeros = jnp.zeros((LANES,), jnp.float32)

        def zbody(i, c):
            cnt[pl.ds(pl.multiple_of(i * LANES, LANES), LANES)] = zeros
            return c

        lax.fori_loop(0, NP // LANES, zbody, 0)
        ones = jnp.full((LANES,), 1.0, jnp.float32)

        def body(i, c):
            idx = dbuf[pl.ds(pl.multiple_of(i * LANES, LANES), LANES)]
            plsc.addupdate_scatter(cnt, [idx], ones)
            return c

        lax.fori_loop(0, epw // LANES, body, 0)
        pltpu.sync_copy(cnt, out_hbm.at[wid])

    return k(dst32)


_KB = 8


def _agg_call(h, sd, zin, split_edges):
    D = 128
    nblk = sd.shape[1]
    nch = (nblk // 2 if split_edges else nblk) * _KB
    rps = NP // NS

    @functools.partial(
        pl.kernel,
        out_type=[jax.ShapeDtypeStruct((NP, D), jnp.float32)] * 2,
        mesh=plsc.VectorSubcoreMesh(**_MESH),
        compiler_params=pltpu.CompilerParams(needs_layout_passes=False),
        scratch_types=[
            pltpu.VMEM((2, 2, _KB, 128), jnp.int32),
            pltpu.VMEM((2, 128, D), jnp.float32),
            pltpu.VMEM_SHARED((NP, D), jnp.float32),
            pltpu.SemaphoreType.DMA,
            pltpu.SemaphoreType.DMA,
            pltpu.SemaphoreType.DMA,
        ],
    )
    def k(h_hbm, sd_hbm, z_hbm, a0_hbm, a1_hbm,
          ibuf, rows, acc, isem, gsem, ssem):
        cid = lax.axis_index("c")
        sid = lax.axis_index("s")

        for t in range(rps // 128):
            pltpu.sync_copy(z_hbm, acc.at[pl.ds(sid * rps + t * 128, 128)])
        plsc.subcore_barrier()

        table = h_hbm if split_edges else h_hbm.at[cid]
        off = cid * nch if split_edges else 0

        def gather_start(par, pb, slot):
            pltpu.async_copy(
                table.at[ibuf.at[pb, 0, slot]], rows.at[par], gsem)

        def gather_wait(par, pb, slot):
            pltpu.make_async_copy(
                table.at[ibuf.at[pb, 0, slot]], rows.at[par], gsem).wait()

        def scatter_wait():
            pltpu.make_async_copy(
                rows.at[0], acc.at[ibuf.at[0, 1, 0]], ssem).wait()

        b0 = (off // _KB if split_edges else 0)
        pltpu.sync_copy(sd_hbm.at[sid, b0], ibuf.at[b0 % 2])
        gather_start(0, b0 % 2, 0)

        def body(t, c):
            g = off + t
            b = g // _KB
            slot = g % _KB
            par = t % 2

            @pl.when(t > 0)
            def _():
                scatter_wait()

            @pl.when(jnp.logical_and(slot == 0, t + _KB < nch))
            def _():
                pltpu.async_copy(
                    sd_hbm.at[sid, b + 1], ibuf.at[(b + 1) % 2], isem)

            nxt = t + 1
            gn = off + nxt
            bn = gn // _KB
            slotn = gn % _KB

            @pl.when(nxt < nch)
            def _():
                @pl.when(jnp.logical_and(slotn == 0, nxt >= _KB))
                def _():
                    pltpu.make_async_copy(
                        sd_hbm.at[sid, bn], ibuf.at[bn % 2], isem).wait()

                gather_start(nxt % 2, bn % 2, slotn)

            gather_wait(par, b % 2, slot)
            pltpu.async_copy(
                rows.at[par], acc.at[ibuf.at[b % 2, 1, slot]],
                ssem, add=True)
            return c

        lax.fori_loop(0, nch, body, 0)
        scatter_wait()
        plsc.subcore_barrier()

        sl = pl.ds(sid * rps, rps)

        @pl.when(cid == 0)
        def _():
            pltpu.sync_copy(acc.at[sl], a0_hbm.at[sl])

        @pl.when(cid == 1)
        def _():
            pltpu.sync_copy(acc.at[sl], a1_hbm.at[sl])

    return k(h, sd, zin)


_NB = 8
_BR = NP // _NB


def _dinv_of(cnt_blk):
    deg = jnp.sum(cnt_blk, axis=0) + 1.0
    return lax.rsqrt(deg)


def _mm1_call(x_pad, cnt, W1):
    def body(x_ref, cnt_ref, w_ref, h_ref):
        dinv = _dinv_of(cnt_ref[...])
        h = jnp.dot(x_ref[...], w_ref[...], preferred_element_type=jnp.float32)
        h = h * dinv[:, None]
        h_ref[0] = h[:, : HID // 2]
        h_ref[1] = h[:, HID // 2:]

    return pl.pallas_call(
        body,
        grid=(_NB,),
        in_specs=[
            pl.BlockSpec((_BR, IN_CH), lambda i: (i, 0)),
            pl.BlockSpec((NC * NS, _BR), lambda i: (0, i)),
            pl.BlockSpec((IN_CH, HID), lambda i: (0, 0)),
        ],
        out_specs=pl.BlockSpec((2, _BR, HID // 2), lambda i: (0, i, 0)),
        out_shape=jax.ShapeDtypeStruct((2, NP, HID // 2), jnp.float32),
    )(x_pad, cnt, W1)


def _mid_call(a0, a1, h1s, cnt, W2, b1):
    def body(a0_ref, a1_ref, h_ref, cnt_ref, w_ref, b_ref, o_ref):
        dinv = _dinv_of(cnt_ref[...])
        x2 = jnp.concatenate(
            [a0_ref[...] + h_ref[0], a1_ref[...] + h_ref[1]], axis=1)
        x2 = jnp.maximum(x2 * dinv[:, None] + b_ref[...], 0.0)
        h2 = jnp.dot(x2, w_ref[...], preferred_element_type=jnp.float32)
        o_ref[...] = h2 * dinv[:, None]

    return pl.pallas_call(
        body,
        grid=(_NB,),
        in_specs=[
            pl.BlockSpec((_BR, HID // 2), lambda i: (i, 0)),
            pl.BlockSpec((_BR, HID // 2), lambda i: (i, 0)),
            pl.BlockSpec((2, _BR, HID // 2), lambda i: (0, i, 0)),
            pl.BlockSpec((NC * NS, _BR), lambda i: (0, i)),
            pl.BlockSpec((HID, OUT), lambda i: (0, 0)),
            pl.BlockSpec((1, HID), lambda i: (0, 0)),
        ],
        out_specs=pl.BlockSpec((_BR, OUT), lambda i: (i, 0)),
        out_shape=jax.ShapeDtypeStruct((NP, OUT), jnp.float32),
    )(a0, a1, h1s, cnt, W2, b1)


def _final_call(p0, p1, h2, cnt, b2):
    def body(p0_ref, p1_ref, h2_ref, cnt_ref, b_ref, o_ref):
        dinv = _dinv_of(cnt_ref[...])
        t = p0_ref[...] + p1_ref[...] + h2_ref[...]
        o_ref[...] = jnp.maximum(t * dinv[:, None] + b_ref[...], 0.0)

    return pl.pallas_call(
        body,
        grid=(_NB,),
        in_specs=[
            pl.BlockSpec((_BR, OUT), lambda i: (i, 0)),
            pl.BlockSpec((_BR, OUT), lambda i: (i, 0)),
            pl.BlockSpec((_BR, OUT), lambda i: (i, 0)),
            pl.BlockSpec((NC * NS, _BR), lambda i: (0, i)),
            pl.BlockSpec((1, OUT), lambda i: (0, 0)),
        ],
        out_specs=pl.BlockSpec((_BR, OUT), lambda i: (i, 0)),
        out_shape=jax.ShapeDtypeStruct((NP, OUT), jnp.float32),
    )(p0, p1, h2, cnt, b2)


def kernel(tensor, edge_index, W1, b1, W2, b2):
    e = edge_index.shape[1]
    gran = NS * _KB * 128
    ep = -(-e // gran) * gran
    npad = ep - e
    ei = edge_index.astype(jnp.int32)
    ar = jnp.arange(npad, dtype=jnp.int32)
    src_p = jnp.concatenate([ei[0], ar % N])
    dst_p = jnp.concatenate([ei[1], N + ar % (NP - N)])
    nblk = ep // NS // (_KB * 128)
    sd = jnp.stack(
        [src_p.reshape(NS, nblk, _KB, 128),
         dst_p.reshape(NS, nblk, _KB, 128)], axis=2)
    dst32 = dst_p.reshape(NC * NS, ep // (NC * NS))
    x_pad = jnp.pad(tensor, ((0, NP - N), (0, 0)))
    z128 = jnp.zeros((128, 128), jnp.float32)

    cnt = _deg_call(dst32)
    h1s = _mm1_call(x_pad, cnt, W1)
    a1a, a1b = _agg_call(h1s, sd, z128, split_edges=False)
    h2 = _mid_call(a1a, a1b, h1s, cnt, W2, b1.reshape(1, -1))
    p0, p1 = _agg_call(h2, sd, z128, split_edges=True)
    out = _final_call(p0, p1, h2, cnt, b2.reshape(1, -1))
    return out[:N]


# speedup vs baseline: 35.5404x; 1.2929x over previous
import functools

import jax
import jax.numpy as jnp
from jax import lax
from jax.experimental import pallas as pl
from jax.experimental.pallas import tpu as pltpu
from jax.experimental.pallas import tpu_sc as plsc

N = 10000
NP = 10240
IN_CH = 128
HID = 256
OUT = 128
NC = 2
NS = 16
LANES = 16

_MESH = dict(core_axis_name="c", subcore_axis_name="s")


def _deg_call(dst32):
    epw = dst32.shape[1]

    @functools.partial(
        pl.kernel,
        out_type=jax.ShapeDtypeStruct((NC * NS, NP), jnp.float32),
        mesh=plsc.VectorSubcoreMesh(**_MESH),
        compiler_params=pltpu.CompilerParams(needs_layout_passes=False),
        scratch_types=[
            pltpu.VMEM((epw,), jnp.int32),
            pltpu.VMEM((NP,), jnp.float32),
        ],
    )
    def k(dst_hbm, out_hbm, dbuf, cnt):
        cid = lax.axis_index("c")
        sid = lax.axis_index("s")
        wid = sid * NC + cid
        pltpu.sync_copy(dst_hbm.at[wid], dbuf)
        zeros = jnp.zeros((LANES,), jnp.float32)

        def zbody(i, c):
            cnt[pl.ds(pl.multiple_of(i * LANES, LANES), LANES)] = zeros
            return c

        lax.fori_loop(0, NP // LANES, zbody, 0)
        ones = jnp.full((LANES,), 1.0, jnp.float32)

        def body(i, c):
            idx = dbuf[pl.ds(pl.multiple_of(i * LANES, LANES), LANES)]
            plsc.addupdate_scatter(cnt, [idx], ones)
            return c

        lax.fori_loop(0, epw // LANES, body, 0)
        pltpu.sync_copy(cnt, out_hbm.at[wid])

    return k(dst32)


_KB = 8


def _agg_call(h, sd, zin):
    D = 128
    nblk = sd.shape[1]
    nch = (nblk // 2) * _KB
    rps = NP // NS

    @functools.partial(
        pl.kernel,
        out_type=[jax.ShapeDtypeStruct((NP, D), jnp.float32)] * 2,
        mesh=plsc.VectorSubcoreMesh(**_MESH),
        compiler_params=pltpu.CompilerParams(needs_layout_passes=False),
        scratch_types=[
            pltpu.VMEM((2, 2, _KB, 128), jnp.int32),
            pltpu.VMEM((2, 128, D), jnp.float32),
            pltpu.VMEM_SHARED((NP, D), jnp.float32),
            pltpu.SemaphoreType.DMA,
            pltpu.SemaphoreType.DMA,
            pltpu.SemaphoreType.DMA,
        ],
    )
    def k(h_hbm, sd_hbm, z_hbm, a0_hbm, a1_hbm,
          ibuf, rows, acc, isem, gsem, ssem):
        cid = lax.axis_index("c")
        sid = lax.axis_index("s")

        for t in range(rps // 128):
            pltpu.sync_copy(z_hbm, acc.at[pl.ds(sid * rps + t * 128, 128)])
        plsc.subcore_barrier()

        table = h_hbm
        off = cid * nch

        def gather_start(par, pb, slot):
            pltpu.async_copy(
                table.at[ibuf.at[pb, 0, slot]], rows.at[par], gsem)

        def gather_wait(par, pb, slot):
            pltpu.make_async_copy(
                table.at[ibuf.at[pb, 0, slot]], rows.at[par], gsem).wait()

        def scatter_wait():
            pltpu.make_async_copy(
                rows.at[0], acc.at[ibuf.at[0, 1, 0]], ssem).wait()

        b0 = off // _KB
        pltpu.sync_copy(sd_hbm.at[sid, b0], ibuf.at[b0 % 2])
        gather_start(0, b0 % 2, 0)

        def body(t, c):
            g = off + t
            b = g // _KB
            slot = g % _KB
            par = t % 2

            @pl.when(t > 0)
            def _():
                scatter_wait()

            @pl.when(jnp.logical_and(slot == 0, t + _KB < nch))
            def _():
                pltpu.async_copy(
                    sd_hbm.at[sid, b + 1], ibuf.at[(b + 1) % 2], isem)

            nxt = t + 1
            gn = off + nxt
            bn = gn // _KB
            slotn = gn % _KB

            @pl.when(nxt < nch)
            def _():
                @pl.when(jnp.logical_and(slotn == 0, nxt >= _KB))
                def _():
                    pltpu.make_async_copy(
                        sd_hbm.at[sid, bn], ibuf.at[bn % 2], isem).wait()

                gather_start(nxt % 2, bn % 2, slotn)

            gather_wait(par, b % 2, slot)
            pltpu.async_copy(
                rows.at[par], acc.at[ibuf.at[b % 2, 1, slot]],
                ssem, add=True)
            return c

        lax.fori_loop(0, nch, body, 0)
        scatter_wait()
        plsc.subcore_barrier()

        sl = pl.ds(sid * rps, rps)

        @pl.when(cid == 0)
        def _():
            pltpu.sync_copy(acc.at[sl], a0_hbm.at[sl])

        @pl.when(cid == 1)
        def _():
            pltpu.sync_copy(acc.at[sl], a1_hbm.at[sl])

    return k(h, sd, zin)


_NB = 8
_BR = NP // _NB


def _dinv_of(cnt_blk):
    deg = jnp.sum(cnt_blk, axis=0) + 1.0
    return lax.rsqrt(deg)


def _xprime_call(x_pad, cnt):
    def body(x_ref, cnt_ref, o_ref):
        dinv = _dinv_of(cnt_ref[...])
        o_ref[...] = x_ref[...] * dinv[:, None]

    return pl.pallas_call(
        body,
        grid=(_NB,),
        in_specs=[
            pl.BlockSpec((_BR, IN_CH), lambda i: (i, 0)),
            pl.BlockSpec((NC * NS, _BR), lambda i: (0, i)),
        ],
        out_specs=pl.BlockSpec((_BR, IN_CH), lambda i: (i, 0)),
        out_shape=jax.ShapeDtypeStruct((NP, IN_CH), jnp.float32),
    )(x_pad, cnt)


def _mid_call(a0, a1, xp, cnt, W1, b1, W2):
    def body(a0_ref, a1_ref, xp_ref, cnt_ref, w1_ref, b_ref, w2_ref, o_ref):
        dinv = _dinv_of(cnt_ref[...])
        t1 = a0_ref[...] + a1_ref[...] + xp_ref[...]
        h1 = jnp.dot(t1, w1_ref[...], preferred_element_type=jnp.float32)
        x2 = jnp.maximum(h1 * dinv[:, None] + b_ref[...], 0.0)
        h2 = jnp.dot(x2, w2_ref[...], preferred_element_type=jnp.float32)
        o_ref[...] = h2 * dinv[:, None]

    return pl.pallas_call(
        body,
        grid=(_NB,),
        in_specs=[
            pl.BlockSpec((_BR, IN_CH), lambda i: (i, 0)),
            pl.BlockSpec((_BR, IN_CH), lambda i: (i, 0)),
            pl.BlockSpec((_BR, IN_CH), lambda i: (i, 0)),
            pl.BlockSpec((NC * NS, _BR), lambda i: (0, i)),
            pl.BlockSpec((IN_CH, HID), lambda i: (0, 0)),
            pl.BlockSpec((1, HID), lambda i: (0, 0)),
            pl.BlockSpec((HID, OUT), lambda i: (0, 0)),
        ],
        out_specs=pl.BlockSpec((_BR, OUT), lambda i: (i, 0)),
        out_shape=jax.ShapeDtypeStruct((NP, OUT), jnp.float32),
    )(a0, a1, xp, cnt, W1, b1, W2)


def _final_call(p0, p1, h2, cnt, b2):
    def body(p0_ref, p1_ref, h2_ref, cnt_ref, b_ref, o_ref):
        dinv = _dinv_of(cnt_ref[...])
        t = p0_ref[...] + p1_ref[...] + h2_ref[...]
        o_ref[...] = jnp.maximum(t * dinv[:, None] + b_ref[...], 0.0)

    return pl.pallas_call(
        body,
        grid=(_NB,),
        in_specs=[
            pl.BlockSpec((_BR, OUT), lambda i: (i, 0)),
            pl.BlockSpec((_BR, OUT), lambda i: (i, 0)),
            pl.BlockSpec((_BR, OUT), lambda i: (i, 0)),
            pl.BlockSpec((NC * NS, _BR), lambda i: (0, i)),
            pl.BlockSpec((1, OUT), lambda i: (0, 0)),
        ],
        out_specs=pl.BlockSpec((_BR, OUT), lambda i: (i, 0)),
        out_shape=jax.ShapeDtypeStruct((NP, OUT), jnp.float32),
    )(p0, p1, h2, cnt, b2)


def kernel(tensor, edge_index, W1, b1, W2, b2):
    e = edge_index.shape[1]
    gran = NS * _KB * 128
    ep = -(-e // gran) * gran
    npad = ep - e
    ei = edge_index.astype(jnp.int32)
    ar = jnp.arange(npad, dtype=jnp.int32)
    src_p = jnp.concatenate([ei[0], ar % N])
    dst_p = jnp.concatenate([ei[1], N + ar % (NP - N)])
    nblk = ep // NS // (_KB * 128)
    sd = jnp.stack(
        [src_p.reshape(NS, nblk, _KB, 128),
         dst_p.reshape(NS, nblk, _KB, 128)], axis=2)
    dst32 = dst_p.reshape(NC * NS, ep // (NC * NS))
    x_pad = jnp.pad(tensor, ((0, NP - N), (0, 0)))
    z128 = jnp.zeros((128, 128), jnp.float32)

    cnt = _deg_call(dst32)
    xp = _xprime_call(x_pad, cnt)
    a0, a1 = _agg_call(xp, sd, z128)
    h2 = _mid_call(a0, a1, xp, cnt, W1, b1.reshape(1, -1), W2)
    p0, p1 = _agg_call(h2, sd, z128)
    out = _final_call(p0, p1, h2, cnt, b2.reshape(1, -1))
    return out[:N]
